# baseline (speedup 1.0000x reference)
import functools

import jax
import jax.numpy as jnp
from jax import lax
from jax.experimental import pallas as pl
from jax.experimental.pallas import tpu as pltpu
from jax.experimental.pallas import tpu_sc as plsc

_NC = 2
_NS = 16
_L = 16
_NBUF = 4
_IB = 128
_SUB = 2
_CHUNK = _IB * _SUB


def _sc_spmm(xtr, vals_p, rc_p, bias, *, n_dst, n_src, hb, n_chunks):
    blocks_per_tile = n_chunks * _SUB
    rows_per_tile = n_dst // _NS
    xrows_per_tile = n_src // _NS
    zrows = 128
    nz_dma = rows_per_tile // zrows
    hq = hb // _L

    mesh = plsc.VectorSubcoreMesh(core_axis_name="c", subcore_axis_name="s")

    @functools.partial(
        pl.kernel,
        out_type=jax.ShapeDtypeStruct((_NC, n_dst, hb), jnp.float32),
        mesh=mesh,
        compiler_params=pltpu.CompilerParams(
            needs_layout_passes=False, use_tc_tiling_on_sc=False),
        scratch_types=[
            pltpu.VMEM_SHARED((n_src, hb), jnp.bfloat16),
            pltpu.VMEM_SHARED((n_dst, hb), jnp.float32),
            pltpu.VMEM((_NBUF, _SUB, _IB), jnp.int32),
            pltpu.VMEM((_NBUF, _SUB, _IB), jnp.int32),
            pltpu.VMEM((_NBUF, _SUB, _IB), jnp.int32),
            pltpu.VMEM((_NBUF, _CHUNK), jnp.float32),
            pltpu.VMEM((_NBUF, _CHUNK, hb), jnp.bfloat16),
            pltpu.VMEM((_NBUF, _CHUNK, hb), jnp.float32),
            pltpu.VMEM((zrows, hb), jnp.float32),
            pltpu.VMEM((rows_per_tile,), jnp.float32),
            pltpu.SemaphoreType.DMA((_NBUF,)),
            pltpu.SemaphoreType.DMA((_NBUF,)),
            pltpu.SemaphoreType.DMA((_NBUF,)),
            pltpu.SemaphoreType.DMA,
        ],
    )
    def k(xtr_hbm, vals_hbm, rc_hbm, bias_hbm, out_hbm,
          xspm, acc, rc_v, cols_v, rows_v, vals_v, gath_v, scl_v, zbuf, bias_v,
          sem_i, sem_g, sem_s, sem_1):
        c = lax.axis_index("c")
        s = lax.axis_index("s")

        xoff = s * xrows_per_tile
        pltpu.sync_copy(xtr_hbm.at[c, pl.ds(xoff, xrows_per_tile)],
                        xspm.at[pl.ds(xoff, xrows_per_tile)])
        pltpu.sync_copy(bias_hbm.at[pl.ds(s * rows_per_tile, rows_per_tile)],
                        bias_v)

        def zb(i, _):
            for q in range(hq):
                zbuf[i, pl.ds(q * _L, _L)] = jnp.zeros((_L,), jnp.float32)
            return 0
        lax.fori_loop(0, zrows, zb, 0)

        def zacc(r, _):
            pltpu.sync_copy(zbuf, acc.at[pl.ds(s * rows_per_tile + r * zrows, zrows)])
            return 0
        lax.fori_loop(0, nz_dma, zacc, 0)
        plsc.subcore_barrier()

        block_tile = s * blocks_per_tile

        def issue_idx(g, b):
            blk = block_tile + (g % n_chunks) * _SUB
            pltpu.async_copy(rc_hbm.at[pl.ds(blk, _SUB)], rc_v.at[b], sem_i.at[b])
            pltpu.async_copy(vals_hbm.at[pl.ds(blk * _IB, _CHUNK)], vals_v.at[b], sem_i.at[b])

        def wait_idx(b):
            pltpu.make_async_copy(rc_hbm.at[pl.ds(0, _SUB)], rc_v.at[b], sem_i.at[b]).wait()
            pltpu.make_async_copy(vals_hbm.at[pl.ds(0, _CHUNK)], vals_v.at[b], sem_i.at[b]).wait()

        def unpack_idx(b):
            mask = jnp.full((_L,), 0xFFFF, jnp.int32)
            sh = jnp.full((_L,), 16, jnp.int32)

            @plsc.parallel_loop(0, _SUB * _IB // _L, unroll=2)
            def _(i):
                j = i // (_IB // _L)
                t = i % (_IB // _L)
                rc16 = rc_v[b, j, pl.ds(t * _L, _L)]
                cols_v[b, j, pl.ds(t * _L, _L)] = rc16 & mask
                rows_v[b, j, pl.ds(t * _L, _L)] = lax.shift_right_logical(rc16, sh)

        def issue_gather(b):
            for j in range(_SUB):
                pltpu.async_copy(xspm.at[cols_v.at[b, j]],
                                 gath_v.at[b, pl.ds(j * _IB, _IB)], sem_g.at[b])

        def wait_gather(b):
            for j in range(_SUB):
                pltpu.make_async_copy(xspm.at[cols_v.at[b, j]],
                                      gath_v.at[b, pl.ds(j * _IB, _IB)],
                                      sem_g.at[b]).wait()

        def issue_scatter(b):
            for j in range(_SUB):
                pltpu.async_copy(scl_v.at[b, pl.ds(j * _IB, _IB)],
                                 acc.at[rows_v.at[b, j]], sem_s.at[b], add=True)

        def wait_scatter(b):
            for j in range(_SUB):
                pltpu.make_async_copy(scl_v.at[b, pl.ds(j * _IB, _IB)],
                                      acc.at[rows_v.at[b, j]], sem_s.at[b]).wait()

        def scale(b):
            @plsc.parallel_loop(0, _CHUNK, unroll=4)
            def _(i):
                vsp = plsc.load_gather(vals_v.at[b], [jnp.full((_L,), i, jnp.int32)])
                for q in range(hb // (2 * _L)):
                    xb = gath_v[b, i, pl.ds(q * 2 * _L, 2 * _L)]
                    u, w = plsc.unpack(xb, format=plsc.PackFormat.INTERLEAVED)
                    scl_v[b, i, pl.ds(q * 2 * _L, _L)] = u * vsp
                    scl_v[b, i, pl.ds(q * 2 * _L + _L, _L)] = w * vsp

        def step(g, b, *, warm):
            bn = (b + 1) % _NBUF
            bp = (b + 2) % _NBUF
            wait_idx(bn)
            unpack_idx(bn)
            issue_gather(bn)
            wait_gather(b)
            scale(b)
            issue_scatter(b)
            if warm:
                wait_scatter(bp)
            issue_idx(g + 2, bp)

        issue_idx(0, 0)
        issue_idx(1, 1)
        wait_idx(0)
        unpack_idx(0)
        issue_gather(0)
        for g in range(4):
            step(g, g, warm=(g >= 2))

        def quad(p, _):
            g0 = p * 4
            for b in range(4):
                step(g0 + b, b, warm=True)
            return 0
        lax.fori_loop(1, n_chunks // 4, quad, 0)

        n = n_chunks
        wait_scatter((n - 2) % _NBUF)
        wait_scatter((n - 1) % _NBUF)
        wait_gather(n % _NBUF)
        wait_idx((n + 1) % _NBUF)

        plsc.subcore_barrier()

        def cp(r, _):
            base = s * rows_per_tile + r * zrows
            pltpu.sync_copy(acc.at[pl.ds(base, zrows)], zbuf)

            @plsc.parallel_loop(0, zrows, unroll=4)
            def _(i):
                bsp = plsc.load_gather(bias_v, [jnp.full((_L,), r * zrows + i, jnp.int32)])
                for q in range(hq):
                    zbuf[i, pl.ds(q * _L, _L)] = zbuf[i, pl.ds(q * _L, _L)] + bsp
            pltpu.sync_copy(zbuf, out_hbm.at[c, pl.ds(base, zrows)])
            return 0
        lax.fori_loop(0, nz_dma, cp, 0)

    return k(xtr, vals_p, rc_p, bias)


def kernel(x, values, bias, rows, cols):
    batch, n_src = x.shape
    n_dst = bias.shape[0]
    nnz = values.shape[0]
    hb = batch // _NC

    perm = jnp.stack([jnp.arange(_L), _L + jnp.arange(_L)], axis=1).reshape(hb)
    xtr = x.reshape(_NC, hb, n_src).transpose(0, 2, 1)[:, :, perm]
    xtr = xtr.astype(jnp.bfloat16)

    e = nnz
    ep_tile = -(-e // _NS)
    n_chunks = 4 * (-(-ep_tile // (4 * _CHUNK)))
    e_pad = _NS * n_chunks * _CHUNK
    pad = e_pad - e
    rc = jnp.pad((rows << 16) | cols, (0, pad)).reshape(e_pad // _IB, _IB)
    vals_p = jnp.pad(values, (0, pad))

    partial = _sc_spmm(xtr, vals_p, rc, bias, n_dst=n_dst, n_src=n_src,
                       hb=hb, n_chunks=n_chunks)
    return partial.transpose(0, 2, 1).reshape(batch, n_dst)


# speedup vs baseline: 1.1550x; 1.1550x over previous
import functools

import jax
import jax.numpy as jnp
from jax import lax
from jax.experimental import pallas as pl
from jax.experimental.pallas import tpu as pltpu
from jax.experimental.pallas import tpu_sc as plsc

_NC = 2
_NS = 16
_L = 16
_NBUF = 4
_IB = 128
_SUB = 3
_CHUNK = _IB * _SUB


def _sc_spmm(xtr, vals_p, rc_p, bias, *, n_dst, n_src, hb, n_chunks):
    blocks_per_tile = n_chunks * _SUB
    rows_per_tile = n_dst // _NS
    xrows_per_tile = n_src // _NS
    zrows = 128
    nz_dma = rows_per_tile // zrows
    hq = hb // _L

    mesh = plsc.VectorSubcoreMesh(core_axis_name="c", subcore_axis_name="s")

    @functools.partial(
        pl.kernel,
        out_type=jax.ShapeDtypeStruct((_NC, n_dst, hb), jnp.float32),
        mesh=mesh,
        compiler_params=pltpu.CompilerParams(
            needs_layout_passes=False, use_tc_tiling_on_sc=False),
        scratch_types=[
            pltpu.VMEM_SHARED((n_src, hb), jnp.float32),
            pltpu.VMEM_SHARED((n_dst, hb), jnp.float32),
            pltpu.VMEM((_NBUF, _SUB, _IB), jnp.int32),
            pltpu.VMEM((_NBUF, _SUB, _IB), jnp.int32),
            pltpu.VMEM((_NBUF, _SUB, _IB), jnp.int32),
            pltpu.VMEM((_NBUF, _CHUNK), jnp.float32),
            pltpu.VMEM((_NBUF, _CHUNK, hb), jnp.float32),
            pltpu.VMEM((zrows, hb), jnp.float32),
            pltpu.VMEM((rows_per_tile,), jnp.float32),
            pltpu.SemaphoreType.DMA((_NBUF,)),
            pltpu.SemaphoreType.DMA((_NBUF,)),
            pltpu.SemaphoreType.DMA((_NBUF,)),
            pltpu.SemaphoreType.DMA,
        ],
    )
    def k(xtr_hbm, vals_hbm, rc_hbm, bias_hbm, out_hbm,
          xspm, acc, rc_v, cols_v, rows_v, vals_v, gath_v, zbuf, bias_v,
          sem_i, sem_g, sem_s, sem_1):
        c = lax.axis_index("c")
        s = lax.axis_index("s")

        xoff = s * xrows_per_tile
        pltpu.sync_copy(xtr_hbm.at[c, pl.ds(xoff, xrows_per_tile)],
                        xspm.at[pl.ds(xoff, xrows_per_tile)])
        pltpu.sync_copy(bias_hbm.at[pl.ds(s * rows_per_tile, rows_per_tile)],
                        bias_v)

        def zb(i, _):
            for q in range(hq):
                zbuf[i, pl.ds(q * _L, _L)] = jnp.zeros((_L,), jnp.float32)
            return 0
        lax.fori_loop(0, zrows, zb, 0)

        def zacc(r, _):
            pltpu.sync_copy(zbuf, acc.at[pl.ds(s * rows_per_tile + r * zrows, zrows)])
            return 0
        lax.fori_loop(0, nz_dma, zacc, 0)
        plsc.subcore_barrier()

        block_tile = s * blocks_per_tile

        def issue_idx(g, b):
            blk = block_tile + (g % n_chunks) * _SUB
            pltpu.async_copy(rc_hbm.at[pl.ds(blk, _SUB)], rc_v.at[b], sem_i.at[b])
            pltpu.async_copy(vals_hbm.at[pl.ds(blk * _IB, _CHUNK)], vals_v.at[b], sem_i.at[b])

        def wait_idx(b):
            pltpu.make_async_copy(rc_hbm.at[pl.ds(0, _SUB)], rc_v.at[b], sem_i.at[b]).wait()
            pltpu.make_async_copy(vals_hbm.at[pl.ds(0, _CHUNK)], vals_v.at[b], sem_i.at[b]).wait()

        def unpack_idx(b):
            mask = jnp.full((_L,), 0xFFFF, jnp.int32)
            sh = jnp.full((_L,), 16, jnp.int32)

            @plsc.parallel_loop(0, _SUB * _IB // _L, unroll=2)
            def _(i):
                j = i // (_IB // _L)
                t = i % (_IB // _L)
                rc16 = rc_v[b, j, pl.ds(t * _L, _L)]
                cols_v[b, j, pl.ds(t * _L, _L)] = rc16 & mask
                rows_v[b, j, pl.ds(t * _L, _L)] = lax.shift_right_logical(rc16, sh)

        def issue_gather(b):
            for j in range(_SUB):
                pltpu.async_copy(xspm.at[cols_v.at[b, j]],
                                 gath_v.at[b, pl.ds(j * _IB, _IB)], sem_g.at[b])

        def wait_gather(b):
            for j in range(_SUB):
                pltpu.make_async_copy(xspm.at[cols_v.at[b, j]],
                                      gath_v.at[b, pl.ds(j * _IB, _IB)],
                                      sem_g.at[b]).wait()

        def issue_scatter(b):
            for j in range(_SUB):
                pltpu.async_copy(gath_v.at[b, pl.ds(j * _IB, _IB)],
                                 acc.at[rows_v.at[b, j]], sem_s.at[b], add=True)

        def wait_scatter(b):
            for j in range(_SUB):
                pltpu.make_async_copy(gath_v.at[b, pl.ds(j * _IB, _IB)],
                                      acc.at[rows_v.at[b, j]], sem_s.at[b]).wait()

        def scale(b):
            @plsc.parallel_loop(0, _CHUNK, unroll=4)
            def _(i):
                vsp = plsc.load_gather(vals_v.at[b], [jnp.full((_L,), i, jnp.int32)])
                for q in range(hq):
                    gath_v[b, i, pl.ds(q * _L, _L)] = (
                        gath_v[b, i, pl.ds(q * _L, _L)] * vsp)

        def step(g, b, *, warm):
            bn = (b + 1) % _NBUF
            bp = (b + 2) % _NBUF
            wait_idx(bn)
            unpack_idx(bn)
            issue_gather(bn)
            wait_gather(b)
            scale(b)
            issue_scatter(b)
            if warm:
                wait_scatter(bp)
            issue_idx(g + 2, bp)

        issue_idx(0, 0)
        issue_idx(1, 1)
        wait_idx(0)
        unpack_idx(0)
        issue_gather(0)
        for g in range(4):
            step(g, g, warm=(g >= 2))

        def quad(p, _):
            g0 = p * 4
            for b in range(4):
                step(g0 + b, b, warm=True)
            return 0
        lax.fori_loop(1, n_chunks // 4, quad, 0)

        n = n_chunks
        wait_scatter((n - 2) % _NBUF)
        wait_scatter((n - 1) % _NBUF)
        wait_gather(n % _NBUF)
        wait_idx((n + 1) % _NBUF)

        plsc.subcore_barrier()

        def cp(r, _):
            base = s * rows_per_tile + r * zrows
            pltpu.sync_copy(acc.at[pl.ds(base, zrows)], zbuf)

            @plsc.parallel_loop(0, zrows, unroll=4)
            def _(i):
                bsp = plsc.load_gather(bias_v, [jnp.full((_L,), r * zrows + i, jnp.int32)])
                for q in range(hq):
                    zbuf[i, pl.ds(q * _L, _L)] = zbuf[i, pl.ds(q * _L, _L)] + bsp
            pltpu.sync_copy(zbuf, out_hbm.at[c, pl.ds(base, zrows)])
            return 0
        lax.fori_loop(0, nz_dma, cp, 0)

    return k(xtr, vals_p, rc_p, bias)


def kernel(x, values, bias, rows, cols):
    batch, n_src = x.shape
    n_dst = bias.shape[0]
    nnz = values.shape[0]
    hb = batch // _NC

    xtr = x.reshape(_NC, hb, n_src).transpose(0, 2, 1)

    e = nnz
    ep_tile = -(-e // _NS)
    n_chunks = 4 * (-(-ep_tile // (4 * _CHUNK)))
    e_pad = _NS * n_chunks * _CHUNK
    pad = e_pad - e
    rc = jnp.pad((rows << 16) | cols, (0, pad)).reshape(e_pad // _IB, _IB)
    vals_p = jnp.pad(values, (0, pad))

    partial = _sc_spmm(xtr, vals_p, rc, bias, n_dst=n_dst, n_src=n_src,
                       hb=hb, n_chunks=n_chunks)
    return partial.transpose(0, 2, 1).reshape(batch, n_dst)


# speedup vs baseline: 1.1943x; 1.0341x over previous
import functools

import jax
import jax.numpy as jnp
from jax import lax
from jax.experimental import pallas as pl
from jax.experimental.pallas import tpu as pltpu
from jax.experimental.pallas import tpu_sc as plsc

_NC = 2
_NS = 16
_L = 16
_NBUF = 4
_IB = 128
_SUB = 3
_CHUNK = _IB * _SUB


def _sc_spmm(xtr, vals_p, rc_p, *, n_dst, n_src, hb, n_chunks):
    blocks_per_tile = n_chunks * _SUB
    rows_per_tile = n_dst // _NS
    xrows_per_tile = n_src // _NS
    zrows = 128
    nz_dma = rows_per_tile // zrows
    hq = hb // _L

    mesh = plsc.VectorSubcoreMesh(core_axis_name="c", subcore_axis_name="s")

    @functools.partial(
        pl.kernel,
        out_type=jax.ShapeDtypeStruct((_NC, n_dst, hb), jnp.float32),
        mesh=mesh,
        compiler_params=pltpu.CompilerParams(
            needs_layout_passes=False, use_tc_tiling_on_sc=False),
        scratch_types=[
            pltpu.VMEM_SHARED((n_src, hb), jnp.float32),
            pltpu.VMEM_SHARED((n_dst, hb), jnp.float32),
            pltpu.VMEM((_NBUF, _SUB, _IB), jnp.int32),
            pltpu.VMEM((_NBUF, _SUB, _IB), jnp.int32),
            pltpu.VMEM((_NBUF, _SUB, _IB), jnp.int32),
            pltpu.VMEM((_NBUF, _CHUNK), jnp.float32),
            pltpu.VMEM((_NBUF, _CHUNK, hb), jnp.float32),
            pltpu.VMEM((zrows, hb), jnp.float32),
            pltpu.SemaphoreType.DMA((_NBUF,)),
            pltpu.SemaphoreType.DMA((_NBUF,)),
            pltpu.SemaphoreType.DMA((_NBUF,)),
            pltpu.SemaphoreType.DMA,
        ],
    )
    def k(xtr_hbm, vals_hbm, rc_hbm, out_hbm,
          xspm, acc, rc_v, cols_v, rows_v, vals_v, gath_v, zbuf,
          sem_i, sem_g, sem_s, sem_1):
        c = lax.axis_index("c")
        s = lax.axis_index("s")

        xoff = s * xrows_per_tile
        stg = pltpu.make_async_copy(xtr_hbm.at[c, pl.ds(xoff, xrows_per_tile)],
                                    xspm.at[pl.ds(xoff, xrows_per_tile)], sem_1)
        stg.start()

        def zb(i, _):
            for q in range(hq):
                zbuf[i, pl.ds(q * _L, _L)] = jnp.zeros((_L,), jnp.float32)
            return 0
        lax.fori_loop(0, zrows, zb, 0)

        for r in range(nz_dma):
            pltpu.async_copy(zbuf, acc.at[pl.ds(s * rows_per_tile + r * zrows, zrows)],
                             sem_1)
        stg.wait()
        for r in range(nz_dma):
            pltpu.make_async_copy(zbuf, acc.at[pl.ds(s * rows_per_tile + r * zrows, zrows)],
                                  sem_1).wait()
        plsc.subcore_barrier()

        block_tile = s * blocks_per_tile

        def issue_idx(g, b):
            blk = block_tile + (g % n_chunks) * _SUB
            pltpu.async_copy(rc_hbm.at[pl.ds(blk, _SUB)], rc_v.at[b], sem_i.at[b])
            pltpu.async_copy(vals_hbm.at[pl.ds(blk * _IB, _CHUNK)], vals_v.at[b], sem_i.at[b])

        def wait_idx(b):
            pltpu.make_async_copy(rc_hbm.at[pl.ds(0, _SUB)], rc_v.at[b], sem_i.at[b]).wait()
            pltpu.make_async_copy(vals_hbm.at[pl.ds(0, _CHUNK)], vals_v.at[b], sem_i.at[b]).wait()

        def unpack_idx(b):
            mask = jnp.full((_L,), 0xFFFF, jnp.int32)
            sh = jnp.full((_L,), 16, jnp.int32)

            @plsc.parallel_loop(0, _SUB * _IB // _L, unroll=2)
            def _(i):
                j = i // (_IB // _L)
                t = i % (_IB // _L)
                rc16 = rc_v[b, j, pl.ds(t * _L, _L)]
                cols_v[b, j, pl.ds(t * _L, _L)] = rc16 & mask
                rows_v[b, j, pl.ds(t * _L, _L)] = lax.shift_right_logical(rc16, sh)

        def issue_gather(b):
            for j in range(_SUB):
                pltpu.async_copy(xspm.at[cols_v.at[b, j]],
                                 gath_v.at[b, pl.ds(j * _IB, _IB)], sem_g.at[b])

        def wait_gather(b):
            for j in range(_SUB):
                pltpu.make_async_copy(xspm.at[cols_v.at[b, j]],
                                      gath_v.at[b, pl.ds(j * _IB, _IB)],
                                      sem_g.at[b]).wait()

        def issue_scatter(b):
            for j in range(_SUB):
                pltpu.async_copy(gath_v.at[b, pl.ds(j * _IB, _IB)],
                                 acc.at[rows_v.at[b, j]], sem_s.at[b], add=True)

        def wait_scatter(b):
            for j in range(_SUB):
                pltpu.make_async_copy(gath_v.at[b, pl.ds(j * _IB, _IB)],
                                      acc.at[rows_v.at[b, j]], sem_s.at[b]).wait()

        def scale(b):
            @plsc.parallel_loop(0, _CHUNK, unroll=4)
            def _(i):
                vsp = plsc.load_gather(vals_v.at[b], [jnp.full((_L,), i, jnp.int32)])
                for q in range(hq):
                    gath_v[b, i, pl.ds(q * _L, _L)] = (
                        gath_v[b, i, pl.ds(q * _L, _L)] * vsp)

        def step(g, b, *, warm):
            bn = (b + 1) % _NBUF
            bp = (b + 2) % _NBUF
            wait_idx(bn)
            unpack_idx(bn)
            issue_gather(bn)
            wait_gather(b)
            scale(b)
            issue_scatter(b)
            if warm:
                wait_scatter(bp)
            issue_idx(g + 2, bp)

        issue_idx(0, 0)
        issue_idx(1, 1)
        wait_idx(0)
        unpack_idx(0)
        issue_gather(0)
        for g in range(4):
            step(g, g, warm=(g >= 2))

        def quad(p, _):
            g0 = p * 4
            for b in range(4):
                step(g0 + b, b, warm=True)
            return 0
        lax.fori_loop(1, n_chunks // 4, quad, 0)

        n = n_chunks
        wait_scatter((n - 2) % _NBUF)
        wait_scatter((n - 1) % _NBUF)
        wait_gather(n % _NBUF)
        wait_idx((n + 1) % _NBUF)

        plsc.subcore_barrier()

        base = s * rows_per_tile
        pltpu.sync_copy(acc.at[pl.ds(base, rows_per_tile)],
                        out_hbm.at[c, pl.ds(base, rows_per_tile)])

    return k(xtr, vals_p, rc_p)


def kernel(x, values, bias, rows, cols):
    batch, n_src = x.shape
    n_dst = bias.shape[0]
    nnz = values.shape[0]
    hb = batch // _NC

    xtr = x.reshape(_NC, hb, n_src).transpose(0, 2, 1)

    e = nnz
    ep_tile = -(-e // _NS)
    n_chunks = 4 * (-(-ep_tile // (4 * _CHUNK)))
    e_pad = _NS * n_chunks * _CHUNK
    pad = e_pad - e
    rc = jnp.pad((rows << 16) | cols, (0, pad)).reshape(e_pad // _IB, _IB)
    vals_p = jnp.pad(values, (0, pad))

    partial = _sc_spmm(xtr, vals_p, rc, n_dst=n_dst, n_src=n_src,
                       hb=hb, n_chunks=n_chunks)
    return partial.transpose(0, 2, 1).reshape(batch, n_dst) + bias[None, :]


# speedup vs baseline: 1.1957x; 1.0012x over previous
import functools

import jax
import jax.numpy as jnp
from jax import lax
from jax.experimental import pallas as pl
from jax.experimental.pallas import tpu as pltpu
from jax.experimental.pallas import tpu_sc as plsc

_NC = 2
_NS = 16
_L = 16
_NBUF = 4
_IB = 128
_SUB = 3
_CHUNK = _IB * _SUB


def _sc_spmm(xtr, vals_p, rows_p, cols_p, *, n_dst, n_src, hb, n_chunks):
    blocks_per_tile = n_chunks * _SUB
    rows_per_tile = n_dst // _NS
    xrows_per_tile = n_src // _NS
    zrows = 128
    nz_dma = rows_per_tile // zrows
    hq = hb // _L

    mesh = plsc.VectorSubcoreMesh(core_axis_name="c", subcore_axis_name="s")

    @functools.partial(
        pl.kernel,
        out_type=jax.ShapeDtypeStruct((_NC, n_dst, hb), jnp.float32),
        mesh=mesh,
        compiler_params=pltpu.CompilerParams(
            needs_layout_passes=False, use_tc_tiling_on_sc=False),
        scratch_types=[
            pltpu.VMEM_SHARED((n_src, hb), jnp.float32),
            pltpu.VMEM_SHARED((n_dst, hb), jnp.float32),
            pltpu.VMEM((_NBUF, _SUB, _IB), jnp.int32),
            pltpu.VMEM((_NBUF, _SUB, _IB), jnp.int32),
            pltpu.VMEM((_NBUF, _CHUNK), jnp.float32),
            pltpu.VMEM((_NBUF, _CHUNK, hb), jnp.float32),
            pltpu.VMEM((zrows, hb), jnp.float32),
            pltpu.SemaphoreType.DMA((_NBUF,)),
            pltpu.SemaphoreType.DMA((_NBUF,)),
            pltpu.SemaphoreType.DMA((_NBUF,)),
            pltpu.SemaphoreType.DMA,
        ],
    )
    def k(xtr_hbm, vals_hbm, rows_hbm, cols_hbm, out_hbm,
          xspm, acc, cols_v, rows_v, vals_v, gath_v, zbuf,
          sem_i, sem_g, sem_s, sem_1):
        c = lax.axis_index("c")
        s = lax.axis_index("s")

        xoff = s * xrows_per_tile
        stg = pltpu.make_async_copy(xtr_hbm.at[c, pl.ds(xoff, xrows_per_tile)],
                                    xspm.at[pl.ds(xoff, xrows_per_tile)], sem_1)
        stg.start()

        def zb(i, _):
            for q in range(hq):
                zbuf[i, pl.ds(q * _L, _L)] = jnp.zeros((_L,), jnp.float32)
            return 0
        lax.fori_loop(0, zrows, zb, 0)

        for r in range(nz_dma):
            pltpu.async_copy(zbuf, acc.at[pl.ds(s * rows_per_tile + r * zrows, zrows)],
                             sem_1)
        stg.wait()
        for r in range(nz_dma):
            pltpu.make_async_copy(zbuf, acc.at[pl.ds(s * rows_per_tile + r * zrows, zrows)],
                                  sem_1).wait()
        plsc.subcore_barrier()

        block_tile = s * blocks_per_tile

        def issue_idx(g, b):
            blk = block_tile + (g % n_chunks) * _SUB
            pltpu.async_copy(cols_hbm.at[pl.ds(blk, _SUB)], cols_v.at[b], sem_i.at[b])
            pltpu.async_copy(rows_hbm.at[pl.ds(blk, _SUB)], rows_v.at[b], sem_i.at[b])
            pltpu.async_copy(vals_hbm.at[pl.ds(blk * _IB, _CHUNK)], vals_v.at[b], sem_i.at[b])

        def wait_idx(b):
            pltpu.make_async_copy(cols_hbm.at[pl.ds(0, _SUB)], cols_v.at[b], sem_i.at[b]).wait()
            pltpu.make_async_copy(rows_hbm.at[pl.ds(0, _SUB)], rows_v.at[b], sem_i.at[b]).wait()
            pltpu.make_async_copy(vals_hbm.at[pl.ds(0, _CHUNK)], vals_v.at[b], sem_i.at[b]).wait()

        def issue_gather(b):
            for j in range(_SUB):
                pltpu.async_copy(xspm.at[cols_v.at[b, j]],
                                 gath_v.at[b, pl.ds(j * _IB, _IB)], sem_g.at[b])

        def wait_gather(b):
            for j in range(_SUB):
                pltpu.make_async_copy(xspm.at[cols_v.at[b, j]],
                                      gath_v.at[b, pl.ds(j * _IB, _IB)],
                                      sem_g.at[b]).wait()

        def issue_scatter(b):
            for j in range(_SUB):
                pltpu.async_copy(gath_v.at[b, pl.ds(j * _IB, _IB)],
                                 acc.at[rows_v.at[b, j]], sem_s.at[b], add=True)

        def wait_scatter(b):
            for j in range(_SUB):
                pltpu.make_async_copy(gath_v.at[b, pl.ds(j * _IB, _IB)],
                                      acc.at[rows_v.at[b, j]], sem_s.at[b]).wait()

        def scale(b):
            @plsc.parallel_loop(0, _CHUNK, unroll=4)
            def _(i):
                vsp = plsc.load_gather(vals_v.at[b], [jnp.full((_L,), i, jnp.int32)])
                for q in range(hq):
                    gath_v[b, i, pl.ds(q * _L, _L)] = (
                        gath_v[b, i, pl.ds(q * _L, _L)] * vsp)

        def step(g, b, *, warm):
            bn = (b + 1) % _NBUF
            bp = (b + 2) % _NBUF
            wait_idx(bn)
            issue_gather(bn)
            wait_gather(b)
            scale(b)
            issue_scatter(b)
            if warm:
                wait_scatter(bp)
            issue_idx(g + 2, bp)

        issue_idx(0, 0)
        issue_idx(1, 1)
        wait_idx(0)
        issue_gather(0)
        for g in range(4):
            step(g, g, warm=(g >= 2))

        def quad(p, _):
            g0 = p * 4
            for b in range(4):
                step(g0 + b, b, warm=True)
            return 0
        lax.fori_loop(1, n_chunks // 4, quad, 0)

        n = n_chunks
        wait_scatter((n - 2) % _NBUF)
        wait_scatter((n - 1) % _NBUF)
        wait_gather(n % _NBUF)
        wait_idx((n + 1) % _NBUF)

        plsc.subcore_barrier()

        base = s * rows_per_tile
        pltpu.sync_copy(acc.at[pl.ds(base, rows_per_tile)],
                        out_hbm.at[c, pl.ds(base, rows_per_tile)])

    return k(xtr, vals_p, rows_p, cols_p)


def kernel(x, values, bias, rows, cols):
    batch, n_src = x.shape
    n_dst = bias.shape[0]
    nnz = values.shape[0]
    hb = batch // _NC

    xtr = x.reshape(_NC, hb, n_src).transpose(0, 2, 1)

    e = nnz
    ep_tile = -(-e // _NS)
    n_chunks = 4 * (-(-ep_tile // (4 * _CHUNK)))
    e_pad = _NS * n_chunks * _CHUNK
    pad = e_pad - e
    rows_p = jnp.pad(rows, (0, pad)).reshape(e_pad // _IB, _IB)
    cols_p = jnp.pad(cols, (0, pad)).reshape(e_pad // _IB, _IB)
    vals_p = jnp.pad(values, (0, pad))

    partial = _sc_spmm(xtr, vals_p, rows_p, cols_p, n_dst=n_dst, n_src=n_src,
                       hb=hb, n_chunks=n_chunks)
    return partial.transpose(0, 2, 1).reshape(batch, n_dst) + bias[None, :]
